# baseline (speedup 1.0000x reference)
import functools
import math

import jax
import jax.numpy as jnp
from jax import lax
from jax.experimental import pallas as pl
from jax.experimental.pallas import tpu as pltpu
from jax.experimental.pallas import tpu_sc as plsc

K = 16
RATIO = 0.25
_relu = jax.nn.relu


def _lrelu(v):
    return jax.nn.leaky_relu(v, 0.01)


def _cdiv(a, b):
    return (a + b - 1) // b


def _dot(a, b):
    return jnp.dot(a, b, preferred_element_type=jnp.float32)


def _row_nb(d, target=16384):
    return max(8, min(512, target // max(d, 1)))


def _linear_body(act, x_ref, w_ref, b_ref, o_ref):
    o = _dot(x_ref[...], w_ref[...]) + b_ref[...]
    o_ref[...] = act(o) if act is not None else o


def _linear(x, p, act):
    n, din = x.shape
    dout = p["W"].shape[1]
    nb = min(_row_nb(max(din, dout)), n)
    return pl.pallas_call(
        functools.partial(_linear_body, act),
        grid=(_cdiv(n, nb),),
        in_specs=[
            pl.BlockSpec((nb, din), lambda i: (i, 0)),
            pl.BlockSpec((din, dout), lambda i: (0, 0)),
            pl.BlockSpec((1, dout), lambda i: (0, 0)),
        ],
        out_specs=pl.BlockSpec((nb, dout), lambda i: (i, 0)),
        out_shape=jax.ShapeDtypeStruct((n, dout), jnp.float32),
    )(x, p["W"], p["b"].reshape(1, dout))


def _qkv_body(x_ref, wi_ref, bi_ref, wq_ref, bq_ref, wk_ref, bk_ref,
              wv_ref, bv_ref, q_ref, k_ref, v_ref):
    x2 = _relu(_dot(x_ref[...], wi_ref[...]) + bi_ref[...])
    q_ref[...] = _dot(x2, wq_ref[...]) + bq_ref[...]
    k_ref[...] = _dot(x2, wk_ref[...]) + bk_ref[...]
    v_ref[...] = _dot(x2, wv_ref[...]) + bv_ref[...]


def _qkv(x, p):
    n, d = x.shape
    nb = min(_row_nb(d), n)
    c = p["conv"]
    full = lambda a: pl.BlockSpec(a.shape, lambda i: (0,) * a.ndim)
    args = []
    for pp in (p["lin_in"], c["lin_src"], c["lin_dst"], c["lin"]):
        args += [pp["W"], pp["b"].reshape(1, -1)]
    return pl.pallas_call(
        _qkv_body,
        grid=(_cdiv(n, nb),),
        in_specs=[pl.BlockSpec((nb, d), lambda i: (i, 0))] + [full(a) for a in args],
        out_specs=[pl.BlockSpec((nb, d), lambda i: (i, 0))] * 3,
        out_shape=[jax.ShapeDtypeStruct((n, d), jnp.float32)] * 3,
    )(x, *args)


def _conv_body(S, d, g_ref, posd_ref, kk_ref,
               pw1_ref, pb1_ref, pw2_ref, pb2_ref,
               aw1_ref, ab1_ref, aw2_ref, ab2_ref,
               wo_ref, bo_ref, o_ref):
    kk = kk_ref[...]
    posd = posd_ref[...]
    pw1, pb1 = pw1_ref[...], pb1_ref[...]
    pw2, pb2 = pw2_ref[...], pb2_ref[...]
    aw1, ab1 = aw1_ref[...], ab1_ref[...]
    aw2, ab2 = aw2_ref[...], ab2_ref[...]
    deltas, alphas, vs = [], [], []
    for j in range(S):
        gj = g_ref[j]
        pdj = posd - gj[:, 2 * d:2 * d + 3]
        h = _lrelu(_dot(pdj, pw1) + pb1)
        dj = _lrelu(_dot(h, pw2) + pb2)
        aj = kk - gj[:, :d] + dj
        a1 = _relu(_dot(aj, aw1) + ab1)
        alphas.append(_relu(_dot(a1, aw2) + ab2))
        deltas.append(dj)
        vs.append(gj[:, d:2 * d])
    amax = alphas[0]
    for j in range(1, S):
        amax = jnp.maximum(amax, alphas[j])
    es = [jnp.exp(a - amax) for a in alphas]
    ssum = es[0]
    for j in range(1, S):
        ssum = ssum + es[j]
    denom = ssum + 1e-16
    acc = (es[0] / denom) * (vs[0] + deltas[0])
    for j in range(1, S):
        acc = acc + (es[j] / denom) * (vs[j] + deltas[j])
    o_ref[...] = _relu(_dot(acc, wo_ref[...]) + bo_ref[...])


def _conv(g, posd, kk, p):
    S, n, D = g.shape
    d = kk.shape[1]
    nb = min(_row_nb(d), n)
    c = p["conv"]
    w = [c["pos_nn"][0]["W"], c["pos_nn"][0]["b"].reshape(1, -1),
         c["pos_nn"][1]["W"], c["pos_nn"][1]["b"].reshape(1, -1),
         c["attn_nn"][0]["W"], c["attn_nn"][0]["b"].reshape(1, -1),
         c["attn_nn"][1]["W"], c["attn_nn"][1]["b"].reshape(1, -1),
         p["lin_out"]["W"], p["lin_out"]["b"].reshape(1, -1)]
    full = lambda a: pl.BlockSpec(a.shape, lambda i: (0,) * a.ndim)
    return pl.pallas_call(
        functools.partial(_conv_body, S, d),
        grid=(_cdiv(n, nb),),
        in_specs=[
            pl.BlockSpec((S, nb, D), lambda i: (0, i, 0)),
            pl.BlockSpec((nb, 3), lambda i: (i, 0)),
            pl.BlockSpec((nb, d), lambda i: (i, 0)),
        ] + [full(a) for a in w],
        out_specs=pl.BlockSpec((nb, d), lambda i: (i, 0)),
        out_shape=jax.ShapeDtypeStruct((n, d), jnp.float32),
    )(g, posd, kk, *w)


_SC_NW = 32


def _sc_gather(table, idx):
    n, D = table.shape
    B = idx.shape[0]
    chunk = max(8, min(128, (400_000 // (D * 4)) // 8 * 8))
    unit = _SC_NW * chunk
    Bpad = _cdiv(B, unit) * unit
    if Bpad != B:
        idx = jnp.concatenate([idx, jnp.zeros((Bpad - B,), jnp.int32)])
    b_per_w = Bpad // _SC_NW
    nchunk = b_per_w // chunk
    mesh = plsc.VectorSubcoreMesh(core_axis_name="c", subcore_axis_name="s")

    @functools.partial(
        pl.kernel, mesh=mesh,
        compiler_params=pltpu.CompilerParams(use_tc_tiling_on_sc=False),
        out_type=jax.ShapeDtypeStruct((Bpad, D), jnp.float32),
        scratch_types=[
            pltpu.VMEM((chunk,), jnp.int32),
            pltpu.VMEM((chunk, D), jnp.float32),
            pltpu.SemaphoreType.DMA,
        ],
    )
    def k(table_hbm, idx_hbm, out_hbm, idx_v, rows_v, sem):
        wid = lax.axis_index("s") * 2 + lax.axis_index("c")
        base = wid * b_per_w

        def body(c, carry):
            off = base + c * chunk
            pltpu.sync_copy(idx_hbm.at[pl.ds(off, chunk)], idx_v)
            pltpu.async_copy(table_hbm.at[idx_v], rows_v, sem).wait()
            pltpu.sync_copy(rows_v, out_hbm.at[pl.ds(off, chunk)])
            return carry

        lax.fori_loop(0, nchunk, body, 0, unroll=False)

    return k(table, idx)[:B]


def _sub_block(p, x, pos, nbr):
    n, d = x.shape
    q, kk, v = _qkv(x, p)
    idx_full = jnp.concatenate([nbr, jnp.arange(n, dtype=jnp.int32)[:, None]], axis=1)
    idxT = idx_full.T
    S = idxT.shape[0]
    tbl = jnp.concatenate([q, v, pos, jnp.zeros((n, 13), jnp.float32)], axis=1)
    rows = _sc_gather(tbl, idxT.reshape(-1)).reshape(S, n, 2 * d + 16)
    return _conv(rows, pos, kk, p)


def _knn_body(nq, nbase, k, exclude_self, qb, with_dist, q_ref, bt_ref, *o_refs):
    i = pl.program_id(0)
    q = q_ref[...]
    bt = bt_ref[...]
    qsq = jnp.sum(q * q, axis=1, keepdims=True)
    bsq = jnp.sum(bt * bt, axis=0, keepdims=True)
    d = qsq - 2.0 * _dot(q, bt) + bsq
    lane = jax.lax.broadcasted_iota(jnp.int32, (qb, nbase), 1)
    if exclude_self:
        rows = i * qb + jax.lax.broadcasted_iota(jnp.int32, (qb, nbase), 0)
        d = jnp.where(lane == rows, jnp.inf, d)
    big = jnp.int32(2**30)
    idx_cols, dist_cols = [], []
    for _ in range(k):
        m = jnp.min(d, axis=1, keepdims=True)
        eq = d == m
        j = jnp.min(jnp.where(eq, lane, big), axis=1, keepdims=True)
        idx_cols.append(j)
        dist_cols.append(m)
        d = jnp.where(eq, jnp.inf, d)
    o_refs[0][...] = jnp.concatenate(idx_cols, axis=1)
    if with_dist:
        o_refs[1][...] = jnp.concatenate(dist_cols, axis=1)


def _knn_pair_body(nq, nbase, k, exclude_self, qb, with_dist,
                   q_ref, bte_ref, bto_ref, *o_refs):
    i = pl.program_id(0)
    nb2 = nbase // 2
    q = q_ref[...]
    bte = bte_ref[...]
    bto = bto_ref[...]
    qsq = jnp.sum(q * q, axis=1, keepdims=True)
    de = qsq - 2.0 * _dot(q, bte) + jnp.sum(bte * bte, axis=0, keepdims=True)
    do = qsq - 2.0 * _dot(q, bto) + jnp.sum(bto * bto, axis=0, keepdims=True)
    lane2 = 2 * jax.lax.broadcasted_iota(jnp.int32, (qb, nb2), 1)
    if exclude_self:
        rows = i * qb + jax.lax.broadcasted_iota(jnp.int32, (qb, nb2), 0)
        de = jnp.where(lane2 == rows, jnp.inf, de)
        do = jnp.where(lane2 + 1 == rows, jnp.inf, do)
    d2 = jnp.minimum(de, do)
    dmax2 = jnp.maximum(de, do)
    p2 = jnp.where(de <= do, 0, 1)
    big = jnp.int32(2**30)
    idx_cols, dist_cols = [], []
    for _ in range(k):
        m = jnp.min(d2, axis=1, keepdims=True)
        eq = d2 == m
        j = jnp.min(jnp.where(eq, lane2 + p2, big), axis=1, keepdims=True)
        idx_cols.append(j)
        dist_cols.append(m)
        d2 = jnp.where(eq, dmax2, d2)
        dmax2 = jnp.where(eq, jnp.inf, dmax2)
        p2 = jnp.where(eq, 1 - p2, p2)
    o_refs[0][...] = jnp.concatenate(idx_cols, axis=1)
    if with_dist:
        o_refs[1][...] = jnp.concatenate(dist_cols, axis=1)


def _knn(query, base, k, exclude_self, with_dist=True):
    nq = query.shape[0]
    nbase = base.shape[0]
    qb = min(256, nq)
    out_specs = [pl.BlockSpec((qb, k), lambda i: (i, 0))]
    out_shape = [jax.ShapeDtypeStruct((nq, k), jnp.int32)]
    if with_dist:
        out_specs.append(pl.BlockSpec((qb, k), lambda i: (i, 0)))
        out_shape.append(jax.ShapeDtypeStruct((nq, k), jnp.float32))
    paired = nbase % 2 == 0 and nbase >= 2048
    if paired:
        nb2 = nbase // 2
        body = functools.partial(_knn_pair_body, nq, nbase, k, exclude_self,
                                 qb, with_dist)
        in_specs = [
            pl.BlockSpec((qb, 3), lambda i: (i, 0)),
            pl.BlockSpec((3, nb2), lambda i: (0, 0)),
            pl.BlockSpec((3, nb2), lambda i: (0, 0)),
        ]
        args = (query, base[0::2].T, base[1::2].T)
    else:
        body = functools.partial(_knn_body, nq, nbase, k, exclude_self,
                                 qb, with_dist)
        in_specs = [
            pl.BlockSpec((qb, 3), lambda i: (i, 0)),
            pl.BlockSpec((3, nbase), lambda i: (0, 0)),
        ]
        args = (query, base.T)
    out = pl.pallas_call(
        body,
        grid=(_cdiv(nq, qb),),
        in_specs=in_specs,
        out_specs=out_specs,
        out_shape=out_shape,
    )(*args)
    return (out[0], out[1]) if with_dist else (out[0], None)


def _fps_body(n, n_sub, c, px_ref, py_ref, pz_ref, o_ref):
    idx2 = (jax.lax.broadcasted_iota(jnp.int32, (8, c), 0) * c
            + jax.lax.broadcasted_iota(jnp.int32, (8, c), 1))
    px, py, pz = px_ref[...], py_ref[...], pz_ref[...]
    o_ref[0] = jnp.int32(0)
    big = jnp.int32(2**30)
    dd0 = jnp.where(idx2 < n, jnp.inf, -jnp.inf)
    lx0 = jnp.sum(jnp.where(idx2 == 0, px, 0.0))
    ly0 = jnp.sum(jnp.where(idx2 == 0, py, 0.0))
    lz0 = jnp.sum(jnp.where(idx2 == 0, pz, 0.0))

    def step(i, carry):
        dd, lx, ly, lz = carry
        d = (px - lx) ** 2 + (py - ly) ** 2 + (pz - lz) ** 2
        dd = jnp.minimum(dd, d)
        m = jnp.max(dd)
        eq = dd == m
        j = jnp.min(jnp.where(eq, idx2, big))
        o_ref[i] = j
        nlx = jnp.sum(jnp.where(eq, px, 0.0))
        nly = jnp.sum(jnp.where(eq, py, 0.0))
        nlz = jnp.sum(jnp.where(eq, pz, 0.0))
        return (dd, nlx, nly, nlz)

    if n_sub > 1:
        jax.lax.fori_loop(1, n_sub, step, (dd0, lx0, ly0, lz0),
                          unroll=False)


def _fps(pos, n_sub):
    n = pos.shape[0]
    c = _cdiv(n, 8)
    pad = 8 * c - n
    coords = []
    for a in range(3):
        v = pos[:, a]
        if pad:
            v = jnp.concatenate([v, jnp.zeros((pad,), jnp.float32)])
        coords.append(v.reshape(8, c))
    return pl.pallas_call(
        functools.partial(_fps_body, n, n_sub, c),
        in_specs=[
            pl.BlockSpec((8, c), lambda: (0, 0)),
            pl.BlockSpec((8, c), lambda: (0, 0)),
            pl.BlockSpec((8, c), lambda: (0, 0)),
        ],
        out_specs=pl.BlockSpec(memory_space=pltpu.SMEM),
        out_shape=jax.ShapeDtypeStruct((n_sub,), jnp.int32),
    )(*coords)


def _pool_body(S, xg_ref, o_ref):
    acc = xg_ref[0]
    for j in range(1, S):
        acc = jnp.maximum(acc, xg_ref[j])
    o_ref[...] = acc


def _pool_max(xg):
    S, n, d = xg.shape
    nb = min(_row_nb(d), n)
    return pl.pallas_call(
        functools.partial(_pool_body, S),
        grid=(_cdiv(n, nb),),
        in_specs=[pl.BlockSpec((S, nb, d), lambda i: (0, i, 0))],
        out_specs=pl.BlockSpec((nb, d), lambda i: (i, 0)),
        out_shape=jax.ShapeDtypeStruct((n, d), jnp.float32),
    )(xg)


def _interp_body(S, xs_ref, w_ref, b_ref, xg_ref, sqd_ref, o_ref):
    sqd = sqd_ref[...]
    wsum = None
    acc = None
    for j in range(S):
        wj = 1.0 / jnp.maximum(jnp.maximum(sqd[:, j:j + 1], 0.0), 1e-16)
        cj = xg_ref[j] * wj
        acc = cj if acc is None else acc + cj
        wsum = wj if wsum is None else wsum + wj
    xi = acc / wsum
    o_ref[...] = _relu(_dot(xs_ref[...], w_ref[...]) + b_ref[...]) + xi


def _interp_up(xs, p_up, xg, sqd):
    S, n, d = xg.shape
    nb = min(_row_nb(d), n)
    return pl.pallas_call(
        functools.partial(_interp_body, S),
        grid=(_cdiv(n, nb),),
        in_specs=[
            pl.BlockSpec((nb, d), lambda i: (i, 0)),
            pl.BlockSpec((d, d), lambda i: (0, 0)),
            pl.BlockSpec((1, d), lambda i: (0, 0)),
            pl.BlockSpec((S, nb, d), lambda i: (0, i, 0)),
            pl.BlockSpec((nb, S), lambda i: (i, 0)),
        ],
        out_specs=pl.BlockSpec((nb, d), lambda i: (i, 0)),
        out_shape=jax.ShapeDtypeStruct((n, d), jnp.float32),
    )(xs, p_up["W"], p_up["b"].reshape(1, d), xg, sqd)


def _head_body(x_ref, w1_ref, b1_ref, w2_ref, b2_ref, o_ref):
    h = _relu(_dot(x_ref[...], w1_ref[...]) + b1_ref[...])
    o = _dot(h, w2_ref[...]) + b2_ref[...]
    m = jnp.max(o, axis=1, keepdims=True)
    e = jnp.exp(o - m)
    o_ref[...] = e / jnp.sum(e, axis=1, keepdims=True)


def _head(x, p0, p1):
    n, d = x.shape
    dh = p0["W"].shape[1]
    do = p1["W"].shape[1]
    nb = min(512, n)
    full = lambda a: pl.BlockSpec(a.shape, lambda i: (0,) * a.ndim)
    args = [p0["W"], p0["b"].reshape(1, dh), p1["W"], p1["b"].reshape(1, do)]
    return pl.pallas_call(
        _head_body,
        grid=(_cdiv(n, nb),),
        in_specs=[pl.BlockSpec((nb, d), lambda i: (i, 0))] + [full(a) for a in args],
        out_specs=pl.BlockSpec((nb, do), lambda i: (i, 0)),
        out_shape=jax.ShapeDtypeStruct((n, do), jnp.float32),
    )(x, *args)


def kernel(x, pos, batch, params):
    nlev = len(params["td"])
    x = _linear(x, params["mlp_input"], _relu)
    nbr, _ = _knn(pos, pos, K, True, with_dist=False)
    x = _sub_block(params["t_in"], x, pos, nbr)
    xs, poss, nbrs = [x], [pos], [nbr]
    for i in range(nlev):
        n = poss[-1].shape[0]
        n_sub = max(int(n * RATIO), 1)
        idxc = _fps(pos, n_sub)
        pos_sub = pos[idxc]
        nbr_pool, _ = _knn(pos_sub, pos, K, False, with_dist=False)
        x = _linear(x, params["down"][i]["mlp"], _relu)
        xg = _sc_gather(x, nbr_pool.T.reshape(-1)).reshape(K, n_sub, x.shape[1])
        x = _pool_max(xg)
        pos = pos_sub
        nbr, _ = _knn(pos, pos, K, True, with_dist=False)
        x = _sub_block(params["td"][i], x, pos, nbr)
        xs.append(x)
        poss.append(pos)
        nbrs.append(nbr)
    x = _linear(x, params["mlp_summit"], _relu)
    x = _sub_block(params["t_summit"], x, pos, nbrs[-1])
    for i in range(nlev):
        up = params["up"][-i - 1]
        x_sub = _linear(x, up["mlp_sub"], _relu)
        idx3, sqd3 = _knn(poss[-i - 2], poss[-i - 1], 3, False)
        np_, dp_ = xs[-i - 2].shape
        xg3 = _sc_gather(x_sub, idx3.T.reshape(-1)).reshape(3, np_, dp_)
        x = _interp_up(xs[-i - 2], up["mlp"], xg3, sqd3)
        x = _sub_block(params["tu"][-i - 1], x, poss[-i - 2], nbrs[-i - 2])
    return _head(x, params["mlp_out"][0], params["mlp_out"][1])


# speedup vs baseline: 1.0130x; 1.0130x over previous
import functools
import math

import jax
import jax.numpy as jnp
from jax import lax
from jax.experimental import pallas as pl
from jax.experimental.pallas import tpu as pltpu
from jax.experimental.pallas import tpu_sc as plsc

K = 16
RATIO = 0.25
_relu = jax.nn.relu


def _lrelu(v):
    return jax.nn.leaky_relu(v, 0.01)


def _cdiv(a, b):
    return (a + b - 1) // b


def _dot(a, b):
    return jnp.dot(a, b, preferred_element_type=jnp.float32)


def _row_nb(d, target=16384):
    return max(8, min(512, target // max(d, 1)))


def _linear_body(act, x_ref, w_ref, b_ref, o_ref):
    o = _dot(x_ref[...], w_ref[...]) + b_ref[...]
    o_ref[...] = act(o) if act is not None else o


def _linear(x, p, act):
    n, din = x.shape
    dout = p["W"].shape[1]
    nb = min(_row_nb(max(din, dout)), n)
    return pl.pallas_call(
        functools.partial(_linear_body, act),
        grid=(_cdiv(n, nb),),
        in_specs=[
            pl.BlockSpec((nb, din), lambda i: (i, 0)),
            pl.BlockSpec((din, dout), lambda i: (0, 0)),
            pl.BlockSpec((1, dout), lambda i: (0, 0)),
        ],
        out_specs=pl.BlockSpec((nb, dout), lambda i: (i, 0)),
        out_shape=jax.ShapeDtypeStruct((n, dout), jnp.float32),
    )(x, p["W"], p["b"].reshape(1, dout))


def _qkv_body(x_ref, wi_ref, bi_ref, wq_ref, bq_ref, wk_ref, bk_ref,
              wv_ref, bv_ref, q_ref, k_ref, v_ref):
    x2 = _relu(_dot(x_ref[...], wi_ref[...]) + bi_ref[...])
    q_ref[...] = _dot(x2, wq_ref[...]) + bq_ref[...]
    k_ref[...] = _dot(x2, wk_ref[...]) + bk_ref[...]
    v_ref[...] = _dot(x2, wv_ref[...]) + bv_ref[...]


def _qkv(x, p):
    n, d = x.shape
    nb = min(_row_nb(d), n)
    c = p["conv"]
    full = lambda a: pl.BlockSpec(a.shape, lambda i: (0,) * a.ndim)
    args = []
    for pp in (p["lin_in"], c["lin_src"], c["lin_dst"], c["lin"]):
        args += [pp["W"], pp["b"].reshape(1, -1)]
    return pl.pallas_call(
        _qkv_body,
        grid=(_cdiv(n, nb),),
        in_specs=[pl.BlockSpec((nb, d), lambda i: (i, 0))] + [full(a) for a in args],
        out_specs=[pl.BlockSpec((nb, d), lambda i: (i, 0))] * 3,
        out_shape=[jax.ShapeDtypeStruct((n, d), jnp.float32)] * 3,
    )(x, *args)


def _conv_body(S, d, g_ref, posd_ref, kk_ref,
               pw1_ref, pb1_ref, pw2_ref, pb2_ref,
               aw1_ref, ab1_ref, aw2_ref, ab2_ref,
               wo_ref, bo_ref, o_ref):
    kk = kk_ref[...]
    posd = posd_ref[...]
    pw1, pb1 = pw1_ref[...], pb1_ref[...]
    pw2, pb2 = pw2_ref[...], pb2_ref[...]
    aw1, ab1 = aw1_ref[...], ab1_ref[...]
    aw2, ab2 = aw2_ref[...], ab2_ref[...]
    deltas, alphas, vs = [], [], []
    for j in range(S):
        gj = g_ref[j]
        pdj = posd - gj[:, 2 * d:2 * d + 3]
        h = _lrelu(_dot(pdj, pw1) + pb1)
        dj = _lrelu(_dot(h, pw2) + pb2)
        aj = kk - gj[:, :d] + dj
        a1 = _relu(_dot(aj, aw1) + ab1)
        alphas.append(_relu(_dot(a1, aw2) + ab2))
        deltas.append(dj)
        vs.append(gj[:, d:2 * d])
    amax = alphas[0]
    for j in range(1, S):
        amax = jnp.maximum(amax, alphas[j])
    es = [jnp.exp(a - amax) for a in alphas]
    ssum = es[0]
    for j in range(1, S):
        ssum = ssum + es[j]
    denom = ssum + 1e-16
    acc = (es[0] / denom) * (vs[0] + deltas[0])
    for j in range(1, S):
        acc = acc + (es[j] / denom) * (vs[j] + deltas[j])
    o_ref[...] = _relu(_dot(acc, wo_ref[...]) + bo_ref[...])


def _conv(g, posd, kk, p):
    S, n, D = g.shape
    d = kk.shape[1]
    nb = min(_row_nb(d), n)
    c = p["conv"]
    w = [c["pos_nn"][0]["W"], c["pos_nn"][0]["b"].reshape(1, -1),
         c["pos_nn"][1]["W"], c["pos_nn"][1]["b"].reshape(1, -1),
         c["attn_nn"][0]["W"], c["attn_nn"][0]["b"].reshape(1, -1),
         c["attn_nn"][1]["W"], c["attn_nn"][1]["b"].reshape(1, -1),
         p["lin_out"]["W"], p["lin_out"]["b"].reshape(1, -1)]
    full = lambda a: pl.BlockSpec(a.shape, lambda i: (0,) * a.ndim)
    return pl.pallas_call(
        functools.partial(_conv_body, S, d),
        grid=(_cdiv(n, nb),),
        in_specs=[
            pl.BlockSpec((S, nb, D), lambda i: (0, i, 0)),
            pl.BlockSpec((nb, 3), lambda i: (i, 0)),
            pl.BlockSpec((nb, d), lambda i: (i, 0)),
        ] + [full(a) for a in w],
        out_specs=pl.BlockSpec((nb, d), lambda i: (i, 0)),
        out_shape=jax.ShapeDtypeStruct((n, d), jnp.float32),
    )(g, posd, kk, *w)


_SC_NW = 32


def _sc_gather(table, idx):
    n, D = table.shape
    B = idx.shape[0]
    chunk = max(8, min(128, (400_000 // (D * 4)) // 8 * 8))
    unit = _SC_NW * chunk
    Bpad = _cdiv(B, unit) * unit
    if Bpad != B:
        idx = jnp.concatenate([idx, jnp.zeros((Bpad - B,), jnp.int32)])
    b_per_w = Bpad // _SC_NW
    nchunk = b_per_w // chunk
    mesh = plsc.VectorSubcoreMesh(core_axis_name="c", subcore_axis_name="s")

    @functools.partial(
        pl.kernel, mesh=mesh,
        compiler_params=pltpu.CompilerParams(use_tc_tiling_on_sc=False),
        out_type=jax.ShapeDtypeStruct((Bpad, D), jnp.float32),
        scratch_types=[
            pltpu.VMEM((chunk,), jnp.int32),
            pltpu.VMEM((chunk, D), jnp.float32),
            pltpu.SemaphoreType.DMA,
        ],
    )
    def k(table_hbm, idx_hbm, out_hbm, idx_v, rows_v, sem):
        wid = lax.axis_index("s") * 2 + lax.axis_index("c")
        base = wid * b_per_w

        def body(c, carry):
            off = base + c * chunk
            pltpu.sync_copy(idx_hbm.at[pl.ds(off, chunk)], idx_v)
            pltpu.async_copy(table_hbm.at[idx_v], rows_v, sem).wait()
            pltpu.sync_copy(rows_v, out_hbm.at[pl.ds(off, chunk)])
            return carry

        lax.fori_loop(0, nchunk, body, 0, unroll=False)

    return k(table, idx)[:B]


def _sub_block(p, x, pos, nbr):
    n, d = x.shape
    q, kk, v = _qkv(x, p)
    idx_full = jnp.concatenate([nbr, jnp.arange(n, dtype=jnp.int32)[:, None]], axis=1)
    idxT = idx_full.T
    S = idxT.shape[0]
    tbl = jnp.concatenate([q, v, pos, jnp.zeros((n, 13), jnp.float32)], axis=1)
    rows = _sc_gather(tbl, idxT.reshape(-1)).reshape(S, n, 2 * d + 16)
    return _conv(rows, pos, kk, p)


def _knn_body(nq, nbase, k, exclude_self, qb, with_dist, q_ref, bt_ref, *o_refs):
    i = pl.program_id(0)
    q = q_ref[...]
    bt = bt_ref[...]
    qsq = jnp.sum(q * q, axis=1, keepdims=True)
    bsq = jnp.sum(bt * bt, axis=0, keepdims=True)
    d = qsq - 2.0 * _dot(q, bt) + bsq
    lane = jax.lax.broadcasted_iota(jnp.int32, (qb, nbase), 1)
    if exclude_self:
        rows = i * qb + jax.lax.broadcasted_iota(jnp.int32, (qb, nbase), 0)
        d = jnp.where(lane == rows, jnp.inf, d)
    big = jnp.int32(2**30)
    idx_cols, dist_cols = [], []
    for _ in range(k):
        m = jnp.min(d, axis=1, keepdims=True)
        eq = d == m
        j = jnp.min(jnp.where(eq, lane, big), axis=1, keepdims=True)
        idx_cols.append(j)
        dist_cols.append(m)
        d = jnp.where(eq, jnp.inf, d)
    o_refs[0][...] = jnp.concatenate(idx_cols, axis=1)
    if with_dist:
        o_refs[1][...] = jnp.concatenate(dist_cols, axis=1)


def _knn(query, base, k, exclude_self, with_dist=True):
    nq = query.shape[0]
    nbase = base.shape[0]
    qb = min(256, nq)
    out_specs = [pl.BlockSpec((qb, k), lambda i: (i, 0))]
    out_shape = [jax.ShapeDtypeStruct((nq, k), jnp.int32)]
    if with_dist:
        out_specs.append(pl.BlockSpec((qb, k), lambda i: (i, 0)))
        out_shape.append(jax.ShapeDtypeStruct((nq, k), jnp.float32))
    body = functools.partial(_knn_body, nq, nbase, k, exclude_self,
                             qb, with_dist)
    in_specs = [
        pl.BlockSpec((qb, 3), lambda i: (i, 0)),
        pl.BlockSpec((3, nbase), lambda i: (0, 0)),
    ]
    args = (query, base.T)
    out = pl.pallas_call(
        body,
        grid=(_cdiv(nq, qb),),
        in_specs=in_specs,
        out_specs=out_specs,
        out_shape=out_shape,
    )(*args)
    return (out[0], out[1]) if with_dist else (out[0], None)


def _fps_body(n, n_sub, c, px_ref, py_ref, pz_ref,
              sx_ref, sy_ref, sz_ref, o_ref):
    idx2 = (jax.lax.broadcasted_iota(jnp.int32, (8, c), 0) * c
            + jax.lax.broadcasted_iota(jnp.int32, (8, c), 1))
    px, py, pz = px_ref[...], py_ref[...], pz_ref[...]
    o_ref[0] = jnp.int32(0)
    big = jnp.int32(2**30)
    dd0 = jnp.where(idx2 < n, jnp.inf, -jnp.inf)

    def step(i, carry):
        dd, lx, ly, lz = carry
        d = (px - lx) ** 2 + (py - ly) ** 2 + (pz - lz) ** 2
        dd = jnp.minimum(dd, d)
        m = jnp.max(dd)
        j = jnp.min(jnp.where(dd == m, idx2, big))
        o_ref[i] = j
        return (dd, sx_ref[j], sy_ref[j], sz_ref[j])

    if n_sub > 1:
        jax.lax.fori_loop(1, n_sub, step,
                          (dd0, sx_ref[0], sy_ref[0], sz_ref[0]),
                          unroll=False)


def _fps(pos, n_sub):
    n = pos.shape[0]
    c = _cdiv(n, 8)
    pad = 8 * c - n
    coords = []
    for a in range(3):
        v = pos[:, a]
        if pad:
            v = jnp.concatenate([v, jnp.zeros((pad,), jnp.float32)])
        coords.append(v.reshape(8, c))
    flat = [pos[:, a] for a in range(3)]
    return pl.pallas_call(
        functools.partial(_fps_body, n, n_sub, c),
        in_specs=[
            pl.BlockSpec((8, c), lambda: (0, 0)),
            pl.BlockSpec((8, c), lambda: (0, 0)),
            pl.BlockSpec((8, c), lambda: (0, 0)),
            pl.BlockSpec(memory_space=pltpu.SMEM),
            pl.BlockSpec(memory_space=pltpu.SMEM),
            pl.BlockSpec(memory_space=pltpu.SMEM),
        ],
        out_specs=pl.BlockSpec(memory_space=pltpu.SMEM),
        out_shape=jax.ShapeDtypeStruct((n_sub,), jnp.int32),
    )(*coords, *flat)


def _pool_body(S, xg_ref, o_ref):
    acc = xg_ref[0]
    for j in range(1, S):
        acc = jnp.maximum(acc, xg_ref[j])
    o_ref[...] = acc


def _pool_max(xg):
    S, n, d = xg.shape
    nb = min(_row_nb(d), n)
    return pl.pallas_call(
        functools.partial(_pool_body, S),
        grid=(_cdiv(n, nb),),
        in_specs=[pl.BlockSpec((S, nb, d), lambda i: (0, i, 0))],
        out_specs=pl.BlockSpec((nb, d), lambda i: (i, 0)),
        out_shape=jax.ShapeDtypeStruct((n, d), jnp.float32),
    )(xg)


def _interp_body(S, xs_ref, w_ref, b_ref, xg_ref, sqd_ref, o_ref):
    sqd = sqd_ref[...]
    wsum = None
    acc = None
    for j in range(S):
        wj = 1.0 / jnp.maximum(jnp.maximum(sqd[:, j:j + 1], 0.0), 1e-16)
        cj = xg_ref[j] * wj
        acc = cj if acc is None else acc + cj
        wsum = wj if wsum is None else wsum + wj
    xi = acc / wsum
    o_ref[...] = _relu(_dot(xs_ref[...], w_ref[...]) + b_ref[...]) + xi


def _interp_up(xs, p_up, xg, sqd):
    S, n, d = xg.shape
    nb = min(_row_nb(d), n)
    return pl.pallas_call(
        functools.partial(_interp_body, S),
        grid=(_cdiv(n, nb),),
        in_specs=[
            pl.BlockSpec((nb, d), lambda i: (i, 0)),
            pl.BlockSpec((d, d), lambda i: (0, 0)),
            pl.BlockSpec((1, d), lambda i: (0, 0)),
            pl.BlockSpec((S, nb, d), lambda i: (0, i, 0)),
            pl.BlockSpec((nb, S), lambda i: (i, 0)),
        ],
        out_specs=pl.BlockSpec((nb, d), lambda i: (i, 0)),
        out_shape=jax.ShapeDtypeStruct((n, d), jnp.float32),
    )(xs, p_up["W"], p_up["b"].reshape(1, d), xg, sqd)


def _head_body(x_ref, w1_ref, b1_ref, w2_ref, b2_ref, o_ref):
    h = _relu(_dot(x_ref[...], w1_ref[...]) + b1_ref[...])
    o = _dot(h, w2_ref[...]) + b2_ref[...]
    m = jnp.max(o, axis=1, keepdims=True)
    e = jnp.exp(o - m)
    o_ref[...] = e / jnp.sum(e, axis=1, keepdims=True)


def _head(x, p0, p1):
    n, d = x.shape
    dh = p0["W"].shape[1]
    do = p1["W"].shape[1]
    nb = min(512, n)
    full = lambda a: pl.BlockSpec(a.shape, lambda i: (0,) * a.ndim)
    args = [p0["W"], p0["b"].reshape(1, dh), p1["W"], p1["b"].reshape(1, do)]
    return pl.pallas_call(
        _head_body,
        grid=(_cdiv(n, nb),),
        in_specs=[pl.BlockSpec((nb, d), lambda i: (i, 0))] + [full(a) for a in args],
        out_specs=pl.BlockSpec((nb, do), lambda i: (i, 0)),
        out_shape=jax.ShapeDtypeStruct((n, do), jnp.float32),
    )(x, *args)


def kernel(x, pos, batch, params):
    nlev = len(params["td"])
    x = _linear(x, params["mlp_input"], _relu)
    nbr, _ = _knn(pos, pos, K, True, with_dist=False)
    x = _sub_block(params["t_in"], x, pos, nbr)
    xs, poss, nbrs = [x], [pos], [nbr]
    for i in range(nlev):
        n = poss[-1].shape[0]
        n_sub = max(int(n * RATIO), 1)
        idxc = _fps(pos, n_sub)
        pos_sub = pos[idxc]
        nbr_pool, _ = _knn(pos_sub, pos, K, False, with_dist=False)
        x = _linear(x, params["down"][i]["mlp"], _relu)
        xg = _sc_gather(x, nbr_pool.T.reshape(-1)).reshape(K, n_sub, x.shape[1])
        x = _pool_max(xg)
        pos = pos_sub
        nbr, _ = _knn(pos, pos, K, True, with_dist=False)
        x = _sub_block(params["td"][i], x, pos, nbr)
        xs.append(x)
        poss.append(pos)
        nbrs.append(nbr)
    x = _linear(x, params["mlp_summit"], _relu)
    x = _sub_block(params["t_summit"], x, pos, nbrs[-1])
    for i in range(nlev):
        up = params["up"][-i - 1]
        x_sub = _linear(x, up["mlp_sub"], _relu)
        idx3, sqd3 = _knn(poss[-i - 2], poss[-i - 1], 3, False)
        np_, dp_ = xs[-i - 2].shape
        xg3 = _sc_gather(x_sub, idx3.T.reshape(-1)).reshape(3, np_, dp_)
        x = _interp_up(xs[-i - 2], up["mlp"], xg3, sqd3)
        x = _sub_block(params["tu"][-i - 1], x, poss[-i - 2], nbrs[-i - 2])
    return _head(x, params["mlp_out"][0], params["mlp_out"][1])
